# baseline (speedup 1.0000x reference)
import functools

import jax
import jax.numpy as jnp
import numpy as np
from jax import lax
from jax.experimental import pallas as pl
from jax.experimental.pallas import tpu as pltpu
from jax.experimental.pallas import tpu_sc as plsc

N_BOXES = 1000
N_PAD = 1024
C = 256
K = 784
NCHUNK = 7
ROWS = 112
BINS_PER_CHUNK = 7
OUT_FLAT = C * 49
TAB_ROWS = 53125
NC, NS = 2, 16
NW = NC * NS
LANES = 16

_HS = (200, 100, 50, 25)
_BASES = (0, 40000, 50000, 52500)
_SCALES = (0.25, 0.125, 0.0625, 0.03125)


def _prep_body(boxes_ref, fy_ref, fx_ref, cy_ref, cx_ref, idx_ref, w_ref):
    x1b = boxes_ref[:, 0:1]
    y1b = boxes_ref[:, 1:2]
    x2b = boxes_ref[:, 2:3]
    y2b = boxes_ref[:, 3:4]
    area = (x2b - x1b) * (y2b - y1b)
    s = jnp.sqrt(area)
    lvl = jnp.floor(4.0 + jnp.log2(s / 224.0) + 1e-6)
    lvl = jnp.clip(lvl, 2.0, 5.0)
    l = (lvl - 2.0).astype(jnp.int32)
    scale = jnp.where(l == 0, _SCALES[0],
                      jnp.where(l == 1, _SCALES[1],
                                jnp.where(l == 2, _SCALES[2], _SCALES[3])))
    H = jnp.where(l == 0, _HS[0],
                  jnp.where(l == 1, _HS[1], jnp.where(l == 2, _HS[2], _HS[3])))
    base = jnp.where(l == 0, _BASES[0],
                     jnp.where(l == 1, _BASES[1],
                               jnp.where(l == 2, _BASES[2], _BASES[3])))
    x1 = x1b * scale
    y1 = y1b * scale
    x2 = x2b * scale
    y2 = y2b * scale
    bin_w = jnp.maximum(x2 - x1, 1.0) / 7.0
    bin_h = jnp.maximum(y2 - y1, 1.0) / 7.0

    fy = fy_ref[0:1, :]
    fx = fx_ref[0:1, :]
    cy = cy_ref[0:1, :]
    cx = cx_ref[0:1, :]

    yy = y1 + fy * bin_h
    xx = x1 + fx * bin_w
    Hf = H.astype(jnp.float32)
    m = ((yy > -1.0) & (yy < Hf) & (xx > -1.0) & (xx < Hf)).astype(jnp.float32)
    yc = jnp.clip(yy, 0.0, Hf - 1.0)
    xc = jnp.clip(xx, 0.0, Hf - 1.0)
    y0f = jnp.floor(yc)
    x0f = jnp.floor(xc)
    y0 = y0f.astype(jnp.int32)
    x0 = x0f.astype(jnp.int32)
    y1i = jnp.minimum(y0 + 1, H - 1)
    x1i = jnp.minimum(x0 + 1, H - 1)
    ly = yc - y0f
    lx = xc - x0f
    wy = jnp.where(cy == 0, 1.0 - ly, ly)
    wx = jnp.where(cx == 0, 1.0 - lx, lx)
    w_ref[...] = wy * wx * m * 0.25
    Y = jnp.where(cy == 0, y0, y1i)
    X = jnp.where(cx == 0, x0, x1i)
    idx_ref[...] = base + Y * H + X


def _prep_idx_w(boxes_p, fy, fx, cy, cx):
    grid = 8
    blk = N_PAD // grid
    return pl.pallas_call(
        _prep_body,
        grid=(grid,),
        in_specs=[
            pl.BlockSpec((blk, 4), lambda i: (i, 0)),
            pl.BlockSpec((8, K), lambda i: (0, 0)),
            pl.BlockSpec((8, K), lambda i: (0, 0)),
            pl.BlockSpec((8, K), lambda i: (0, 0)),
            pl.BlockSpec((8, K), lambda i: (0, 0)),
        ],
        out_specs=[
            pl.BlockSpec((blk, K), lambda i: (i, 0)),
            pl.BlockSpec((blk, K), lambda i: (i, 0)),
        ],
        out_shape=[
            jax.ShapeDtypeStruct((N_PAD, K), jnp.int32),
            jax.ShapeDtypeStruct((N_PAD, K), jnp.float32),
        ],
    )(boxes_p, fy, fx, cy, cx)


def _sc_body(table_hbm, idx_hbm, w_hbm, out_hbm, idx_v, w_v, rows_v, out_v, sem):
    wid = lax.axis_index("s") * NC + lax.axis_index("c")
    nb = jnp.where(wid < N_BOXES - 31 * NW, 32, 31)

    @pl.loop(0, nb)
    def _box(i):
        n = i * NW + wid
        pltpu.sync_copy(idx_hbm.at[n], idx_v)
        pltpu.sync_copy(w_hbm.at[n], w_v)
        pltpu.async_copy(table_hbm.at[idx_v.at[0]], rows_v.at[0], sem.at[0])

        @pl.loop(0, NCHUNK)
        def _chunk(c):
            s = c % 2

            @pl.when(c + 1 < NCHUNK)
            def _prefetch():
                pltpu.async_copy(table_hbm.at[idx_v.at[c + 1]],
                                 rows_v.at[(c + 1) % 2], sem.at[(c + 1) % 2])

            pltpu.make_async_copy(table_hbm.at[idx_v.at[c]], rows_v.at[s],
                                  sem.at[s]).wait()

            @pl.loop(0, BINS_PER_CHUNK)
            def _bin(lb):
                g = c * BINS_PER_CHUNK + lb
                kbase = c * ROWS + lb * 16
                w16 = w_v[pl.ds(kbase, LANES)]
                accs = [jnp.zeros((LANES,), jnp.float32) for _ in range(16)]
                for j in range(16):
                    wvec = jnp.take_along_axis(
                        w16, jnp.full((LANES,), j, jnp.int32), axis=0,
                        mode="promise_in_bounds")
                    for t2 in range(8):
                        xi = rows_v[s, lb * 16 + j, pl.ds(16 * t2, 16)]
                        a = lax.bitcast_convert_type(xi << 16, jnp.float32)
                        b = lax.bitcast_convert_type(xi & jnp.int32(-65536),
                                                     jnp.float32)
                        accs[t2] = accs[t2] + wvec * a
                        accs[8 + t2] = accs[8 + t2] + wvec * b
                for t2 in range(8):
                    lo_b = lax.bitcast_convert_type(accs[t2], jnp.int32)
                    lo_r = (lo_b + (32767 + ((lo_b >> 16) & 1))) >> 16
                    hi_b = lax.bitcast_convert_type(accs[8 + t2], jnp.int32)
                    hi_r = (hi_b + (32767 + ((hi_b >> 16) & 1))) >> 16
                    out_v[g, pl.ds(16 * t2, LANES)] = (
                        (lo_r & jnp.int32(65535)) | (hi_r << 16))

        pltpu.sync_copy(out_v, out_hbm.at[n])


def _sc_gather(table, idx3, w2):
    mesh = plsc.VectorSubcoreMesh(core_axis_name="c", subcore_axis_name="s")
    fn = functools.partial(
        pl.kernel,
        out_type=jax.ShapeDtypeStruct((N_BOXES, 49, C // 2), jnp.int32),
        mesh=mesh,
        scratch_types=[
            pltpu.VMEM((NCHUNK, ROWS), jnp.int32),
            pltpu.VMEM((K,), jnp.float32),
            pltpu.VMEM((2, ROWS, C // 2), jnp.int32),
            pltpu.VMEM((49, C // 2), jnp.int32),
            pltpu.SemaphoreType.DMA((2,)),
        ],
    )(_sc_body)
    return fn(table, idx3, w2)


def _static_grids():
    k = np.arange(K)
    b = k // 16
    py = b // 7
    px = b % 7
    j = k % 16
    samp = j // 4
    iy = samp // 2
    ix = samp % 2
    corner = j % 4
    cy = corner // 2
    cx = corner % 2
    fy = (py + (iy + 0.5) / 2.0).astype(np.float32)
    fx = (px + (ix + 0.5) / 2.0).astype(np.float32)
    tile = lambda a: jnp.asarray(np.tile(a[None, :], (8, 1)))
    return (tile(fy), tile(fx), tile(cy.astype(np.int32)),
            tile(cx.astype(np.int32)))


@jax.jit
def kernel(feat0, feat1, feat2, feat3, boxes):
    table = jnp.concatenate(
        [jnp.transpose(f[0], (1, 2, 0)).reshape(-1, C)
         for f in (feat0, feat1, feat2, feat3)], axis=0)
    lo = jax.lax.bitcast_convert_type(
        table[:, :128].astype(jnp.bfloat16), jnp.uint16).astype(jnp.uint32)
    hi = jax.lax.bitcast_convert_type(
        table[:, 128:].astype(jnp.bfloat16), jnp.uint16).astype(jnp.uint32)
    table = jax.lax.bitcast_convert_type(lo | (hi << 16), jnp.int32)
    boxes_p = jnp.zeros((N_PAD, 4), jnp.float32).at[:N_BOXES].set(boxes)
    fy, fx, cy, cx = _static_grids()
    idx, w = _prep_idx_w(boxes_p, fy, fx, cy, cx)
    idx3 = idx[:N_BOXES].reshape(N_BOXES, NCHUNK, ROWS)
    w2 = w[:N_BOXES]
    ow = _sc_gather(table, idx3, w2)
    lo_f = jax.lax.bitcast_convert_type(ow << 16, jnp.float32)
    hi_f = jax.lax.bitcast_convert_type(ow & jnp.int32(-65536), jnp.float32)
    out = jnp.concatenate([lo_f, hi_f], axis=2)
    return jnp.transpose(out, (0, 2, 1)).reshape(N_BOXES, C, 7, 7)


# speedup vs baseline: 1.0772x; 1.0772x over previous
import functools

import jax
import jax.numpy as jnp
import numpy as np
from jax import lax
from jax.experimental import pallas as pl
from jax.experimental.pallas import tpu as pltpu
from jax.experimental.pallas import tpu_sc as plsc

N_BOXES = 1000
N_PAD = 1024
C = 256
K = 784
NCHUNK = 7
ROWS = 112
BINS_PER_CHUNK = 7
OUT_FLAT = C * 49
TAB_ROWS = 53125
NC, NS = 2, 16
NW = NC * NS
LANES = 16

_HS = (200, 100, 50, 25)
_BASES = (0, 40000, 50000, 52500)
_SCALES = (0.25, 0.125, 0.0625, 0.03125)


def _prep_body(boxes_ref, fy_ref, fx_ref, cy_ref, cx_ref, idx_ref, w_ref):
    x1b = boxes_ref[:, 0:1]
    y1b = boxes_ref[:, 1:2]
    x2b = boxes_ref[:, 2:3]
    y2b = boxes_ref[:, 3:4]
    area = (x2b - x1b) * (y2b - y1b)
    s = jnp.sqrt(area)
    lvl = jnp.floor(4.0 + jnp.log2(s / 224.0) + 1e-6)
    lvl = jnp.clip(lvl, 2.0, 5.0)
    l = (lvl - 2.0).astype(jnp.int32)
    scale = jnp.where(l == 0, _SCALES[0],
                      jnp.where(l == 1, _SCALES[1],
                                jnp.where(l == 2, _SCALES[2], _SCALES[3])))
    H = jnp.where(l == 0, _HS[0],
                  jnp.where(l == 1, _HS[1], jnp.where(l == 2, _HS[2], _HS[3])))
    base = jnp.where(l == 0, _BASES[0],
                     jnp.where(l == 1, _BASES[1],
                               jnp.where(l == 2, _BASES[2], _BASES[3])))
    x1 = x1b * scale
    y1 = y1b * scale
    x2 = x2b * scale
    y2 = y2b * scale
    bin_w = jnp.maximum(x2 - x1, 1.0) / 7.0
    bin_h = jnp.maximum(y2 - y1, 1.0) / 7.0

    fy = fy_ref[0:1, :]
    fx = fx_ref[0:1, :]
    cy = cy_ref[0:1, :]
    cx = cx_ref[0:1, :]

    yy = y1 + fy * bin_h
    xx = x1 + fx * bin_w
    Hf = H.astype(jnp.float32)
    m = ((yy > -1.0) & (yy < Hf) & (xx > -1.0) & (xx < Hf)).astype(jnp.float32)
    yc = jnp.clip(yy, 0.0, Hf - 1.0)
    xc = jnp.clip(xx, 0.0, Hf - 1.0)
    y0f = jnp.floor(yc)
    x0f = jnp.floor(xc)
    y0 = y0f.astype(jnp.int32)
    x0 = x0f.astype(jnp.int32)
    y1i = jnp.minimum(y0 + 1, H - 1)
    x1i = jnp.minimum(x0 + 1, H - 1)
    ly = yc - y0f
    lx = xc - x0f
    wy = jnp.where(cy == 0, 1.0 - ly, ly)
    wx = jnp.where(cx == 0, 1.0 - lx, lx)
    w_ref[...] = wy * wx * m * 0.25
    Y = jnp.where(cy == 0, y0, y1i)
    X = jnp.where(cx == 0, x0, x1i)
    idx_ref[...] = base + Y * H + X


def _prep_idx_w(boxes_p, fy, fx, cy, cx):
    grid = 8
    blk = N_PAD // grid
    return pl.pallas_call(
        _prep_body,
        grid=(grid,),
        in_specs=[
            pl.BlockSpec((blk, 4), lambda i: (i, 0)),
            pl.BlockSpec((8, K), lambda i: (0, 0)),
            pl.BlockSpec((8, K), lambda i: (0, 0)),
            pl.BlockSpec((8, K), lambda i: (0, 0)),
            pl.BlockSpec((8, K), lambda i: (0, 0)),
        ],
        out_specs=[
            pl.BlockSpec((blk, K), lambda i: (i, 0)),
            pl.BlockSpec((blk, K), lambda i: (i, 0)),
        ],
        out_shape=[
            jax.ShapeDtypeStruct((N_PAD, K), jnp.int32),
            jax.ShapeDtypeStruct((N_PAD, K), jnp.float32),
        ],
    )(boxes_p, fy, fx, cy, cx)


def _sc_body(table_hbm, idx_hbm, w_hbm, out_hbm, idx_v, w_v, rows_v, out_v, sem):
    wid = lax.axis_index("s") * NC + lax.axis_index("c")
    nb = jnp.where(wid < N_BOXES - 31 * NW, 32, 31)

    @pl.loop(0, nb)
    def _box(i):
        n = i * NW + wid
        pltpu.sync_copy(idx_hbm.at[n], idx_v)
        pltpu.sync_copy(w_hbm.at[n], w_v)
        pltpu.async_copy(table_hbm.at[idx_v.at[0]], rows_v.at[0], sem.at[0])

        @pl.loop(0, NCHUNK)
        def _chunk(c):
            s = c % 2

            @pl.when(c + 1 < NCHUNK)
            def _prefetch():
                pltpu.async_copy(table_hbm.at[idx_v.at[c + 1]],
                                 rows_v.at[(c + 1) % 2], sem.at[(c + 1) % 2])

            pltpu.make_async_copy(table_hbm.at[idx_v.at[c]], rows_v.at[s],
                                  sem.at[s]).wait()

            @pl.loop(0, BINS_PER_CHUNK)
            def _bin(lb):
                g = c * BINS_PER_CHUNK + lb
                kbase = c * ROWS + lb * 16
                w16 = w_v[pl.ds(kbase, LANES)]
                accs = [jnp.zeros((LANES,), jnp.float32) for _ in range(16)]
                for j in range(16):
                    wvec = jnp.take_along_axis(
                        w16, jnp.full((LANES,), j, jnp.int32), axis=0,
                        mode="promise_in_bounds")
                    for t2 in range(8):
                        xi = rows_v[s, lb * 16 + j, pl.ds(16 * t2, 16)]
                        a = lax.bitcast_convert_type(xi << 16, jnp.float32)
                        b = lax.bitcast_convert_type(xi & jnp.int32(-65536),
                                                     jnp.float32)
                        accs[t2] = accs[t2] + wvec * a
                        accs[8 + t2] = accs[8 + t2] + wvec * b
                for t in range(16):
                    out_v[g, pl.ds(16 * t, LANES)] = accs[t]

        pltpu.sync_copy(out_v, out_hbm.at[n])


def _sc_gather(table, idx3, w2):
    mesh = plsc.VectorSubcoreMesh(core_axis_name="c", subcore_axis_name="s")
    fn = functools.partial(
        pl.kernel,
        out_type=jax.ShapeDtypeStruct((N_BOXES, 49, C), jnp.float32),
        mesh=mesh,
        scratch_types=[
            pltpu.VMEM((NCHUNK, ROWS), jnp.int32),
            pltpu.VMEM((K,), jnp.float32),
            pltpu.VMEM((2, ROWS, C // 2), jnp.int32),
            pltpu.VMEM((49, C), jnp.float32),
            pltpu.SemaphoreType.DMA((2,)),
        ],
    )(_sc_body)
    return fn(table, idx3, w2)


def _static_grids():
    k = np.arange(K)
    b = k // 16
    py = b // 7
    px = b % 7
    j = k % 16
    samp = j // 4
    iy = samp // 2
    ix = samp % 2
    corner = j % 4
    cy = corner // 2
    cx = corner % 2
    fy = (py + (iy + 0.5) / 2.0).astype(np.float32)
    fx = (px + (ix + 0.5) / 2.0).astype(np.float32)
    tile = lambda a: jnp.asarray(np.tile(a[None, :], (8, 1)))
    return (tile(fy), tile(fx), tile(cy.astype(np.int32)),
            tile(cx.astype(np.int32)))


@jax.jit
def kernel(feat0, feat1, feat2, feat3, boxes):
    table = jnp.concatenate(
        [jnp.transpose(f[0], (1, 2, 0)).reshape(-1, C)
         for f in (feat0, feat1, feat2, feat3)], axis=0)
    lo = jax.lax.bitcast_convert_type(
        table[:, :128].astype(jnp.bfloat16), jnp.uint16).astype(jnp.uint32)
    hi = jax.lax.bitcast_convert_type(
        table[:, 128:].astype(jnp.bfloat16), jnp.uint16).astype(jnp.uint32)
    table = jax.lax.bitcast_convert_type(lo | (hi << 16), jnp.int32)
    boxes_p = jnp.zeros((N_PAD, 4), jnp.float32).at[:N_BOXES].set(boxes)
    fy, fx, cy, cx = _static_grids()
    idx, w = _prep_idx_w(boxes_p, fy, fx, cy, cx)
    idx3 = idx[:N_BOXES].reshape(N_BOXES, NCHUNK, ROWS)
    w2 = w[:N_BOXES]
    out = _sc_gather(table, idx3, w2).reshape(N_BOXES, 49, C)
    return jnp.transpose(out, (0, 2, 1)).reshape(N_BOXES, C, 7, 7)


# speedup vs baseline: 1.1022x; 1.0232x over previous
import functools

import jax
import jax.numpy as jnp
import numpy as np
from jax import lax
from jax.experimental import pallas as pl
from jax.experimental.pallas import tpu as pltpu
from jax.experimental.pallas import tpu_sc as plsc

N_BOXES = 1000
N_PAD = 1024
C = 256
K = 784
NCHUNK = 7
ROWS = 112
BINS_PER_CHUNK = 7
OUT_FLAT = C * 49
TAB_ROWS = 53125
NC, NS = 2, 16
NW = NC * NS
LANES = 16

_HS = (200, 100, 50, 25)
_BASES = (0, 40000, 50000, 52500)
_SCALES = (0.25, 0.125, 0.0625, 0.03125)


def _prep_body(boxes_ref, fy_ref, fx_ref, cy_ref, cx_ref, idx_ref, w_ref):
    x1b = boxes_ref[:, 0:1]
    y1b = boxes_ref[:, 1:2]
    x2b = boxes_ref[:, 2:3]
    y2b = boxes_ref[:, 3:4]
    area = (x2b - x1b) * (y2b - y1b)
    s = jnp.sqrt(area)
    lvl = jnp.floor(4.0 + jnp.log2(s / 224.0) + 1e-6)
    lvl = jnp.clip(lvl, 2.0, 5.0)
    l = (lvl - 2.0).astype(jnp.int32)
    scale = jnp.where(l == 0, _SCALES[0],
                      jnp.where(l == 1, _SCALES[1],
                                jnp.where(l == 2, _SCALES[2], _SCALES[3])))
    H = jnp.where(l == 0, _HS[0],
                  jnp.where(l == 1, _HS[1], jnp.where(l == 2, _HS[2], _HS[3])))
    base = jnp.where(l == 0, _BASES[0],
                     jnp.where(l == 1, _BASES[1],
                               jnp.where(l == 2, _BASES[2], _BASES[3])))
    x1 = x1b * scale
    y1 = y1b * scale
    x2 = x2b * scale
    y2 = y2b * scale
    bin_w = jnp.maximum(x2 - x1, 1.0) / 7.0
    bin_h = jnp.maximum(y2 - y1, 1.0) / 7.0

    fy = fy_ref[0:1, :]
    fx = fx_ref[0:1, :]
    cy = cy_ref[0:1, :]
    cx = cx_ref[0:1, :]

    yy = y1 + fy * bin_h
    xx = x1 + fx * bin_w
    Hf = H.astype(jnp.float32)
    m = ((yy > -1.0) & (yy < Hf) & (xx > -1.0) & (xx < Hf)).astype(jnp.float32)
    yc = jnp.clip(yy, 0.0, Hf - 1.0)
    xc = jnp.clip(xx, 0.0, Hf - 1.0)
    y0f = jnp.floor(yc)
    x0f = jnp.floor(xc)
    y0 = y0f.astype(jnp.int32)
    x0 = x0f.astype(jnp.int32)
    y1i = jnp.minimum(y0 + 1, H - 1)
    x1i = jnp.minimum(x0 + 1, H - 1)
    ly = yc - y0f
    lx = xc - x0f
    wy = jnp.where(cy == 0, 1.0 - ly, ly)
    wx = jnp.where(cx == 0, 1.0 - lx, lx)
    w_ref[...] = wy * wx * m * 0.25
    Y = jnp.where(cy == 0, y0, y1i)
    X = jnp.where(cx == 0, x0, x1i)
    idx_ref[...] = base + Y * H + X


def _prep_idx_w(boxes_p, fy, fx, cy, cx):
    grid = 8
    blk = N_PAD // grid
    return pl.pallas_call(
        _prep_body,
        grid=(grid,),
        in_specs=[
            pl.BlockSpec((blk, 4), lambda i: (i, 0)),
            pl.BlockSpec((8, K), lambda i: (0, 0)),
            pl.BlockSpec((8, K), lambda i: (0, 0)),
            pl.BlockSpec((8, K), lambda i: (0, 0)),
            pl.BlockSpec((8, K), lambda i: (0, 0)),
        ],
        out_specs=[
            pl.BlockSpec((blk, K), lambda i: (i, 0)),
            pl.BlockSpec((blk, K), lambda i: (i, 0)),
        ],
        out_shape=[
            jax.ShapeDtypeStruct((N_PAD, K), jnp.int32),
            jax.ShapeDtypeStruct((N_PAD, K), jnp.float32),
        ],
    )(boxes_p, fy, fx, cy, cx)


def _sc_body(table_hbm, idx_hbm, w_hbm, out_hbm, idx_v, w_v, rows_v, out_v, sem):
    wid = lax.axis_index("s") * NC + lax.axis_index("c")
    nb = jnp.where(wid < N_BOXES - 31 * NW, 32, 31)

    @pl.loop(0, nb)
    def _box(i):
        n = i * NW + wid
        pltpu.sync_copy(idx_hbm.at[n], idx_v)
        pltpu.sync_copy(w_hbm.at[n], w_v)
        pltpu.async_copy(table_hbm.at[idx_v.at[0]], rows_v.at[0], sem.at[0])

        @pl.loop(0, NCHUNK)
        def _chunk(c):
            s = c % 2

            @pl.when(c + 1 < NCHUNK)
            def _prefetch():
                pltpu.async_copy(table_hbm.at[idx_v.at[c + 1]],
                                 rows_v.at[(c + 1) % 2], sem.at[(c + 1) % 2])

            pltpu.make_async_copy(table_hbm.at[idx_v.at[c]], rows_v.at[s],
                                  sem.at[s]).wait()

            @pl.loop(0, BINS_PER_CHUNK)
            def _bin(lb):
                g = c * BINS_PER_CHUNK + lb
                kbase = c * ROWS + lb * 16
                w16 = w_v[pl.ds(kbase, LANES)]
                accs = [jnp.zeros((LANES,), jnp.float32) for _ in range(16)]
                for j in range(16):
                    wvec = jnp.take_along_axis(
                        w16, jnp.full((LANES,), j, jnp.int32), axis=0,
                        mode="promise_in_bounds")
                    for t2 in range(8):
                        xi = rows_v[s, lb * 16 + j, pl.ds(16 * t2, 16)]
                        a = lax.bitcast_convert_type(xi << 16, jnp.float32)
                        b = lax.bitcast_convert_type(xi, jnp.float32)
                        accs[t2] = accs[t2] + wvec * a
                        accs[8 + t2] = accs[8 + t2] + wvec * b
                for t in range(16):
                    out_v[g, pl.ds(16 * t, LANES)] = accs[t]

        pltpu.sync_copy(out_v, out_hbm.at[n])


def _sc_gather(table, idx3, w2):
    mesh = plsc.VectorSubcoreMesh(core_axis_name="c", subcore_axis_name="s")
    fn = functools.partial(
        pl.kernel,
        out_type=jax.ShapeDtypeStruct((N_BOXES, 49, C), jnp.float32),
        mesh=mesh,
        scratch_types=[
            pltpu.VMEM((NCHUNK, ROWS), jnp.int32),
            pltpu.VMEM((K,), jnp.float32),
            pltpu.VMEM((2, ROWS, C // 2), jnp.int32),
            pltpu.VMEM((49, C), jnp.float32),
            pltpu.SemaphoreType.DMA((2,)),
        ],
    )(_sc_body)
    return fn(table, idx3, w2)


def _static_grids():
    k = np.arange(K)
    b = k // 16
    py = b // 7
    px = b % 7
    j = k % 16
    samp = j // 4
    iy = samp // 2
    ix = samp % 2
    corner = j % 4
    cy = corner // 2
    cx = corner % 2
    fy = (py + (iy + 0.5) / 2.0).astype(np.float32)
    fx = (px + (ix + 0.5) / 2.0).astype(np.float32)
    tile = lambda a: jnp.asarray(np.tile(a[None, :], (8, 1)))
    return (tile(fy), tile(fx), tile(cy.astype(np.int32)),
            tile(cx.astype(np.int32)))


@jax.jit
def kernel(feat0, feat1, feat2, feat3, boxes):
    table = jnp.concatenate(
        [jnp.transpose(f[0], (1, 2, 0)).reshape(-1, C)
         for f in (feat0, feat1, feat2, feat3)], axis=0)
    lo = jax.lax.bitcast_convert_type(
        table[:, :128].astype(jnp.bfloat16), jnp.uint16).astype(jnp.uint32)
    hi = jax.lax.bitcast_convert_type(
        table[:, 128:].astype(jnp.bfloat16), jnp.uint16).astype(jnp.uint32)
    table = jax.lax.bitcast_convert_type(lo | (hi << 16), jnp.int32)
    boxes_p = jnp.zeros((N_PAD, 4), jnp.float32).at[:N_BOXES].set(boxes)
    fy, fx, cy, cx = _static_grids()
    idx, w = _prep_idx_w(boxes_p, fy, fx, cy, cx)
    idx3 = idx[:N_BOXES].reshape(N_BOXES, NCHUNK, ROWS)
    w2 = w[:N_BOXES]
    out = _sc_gather(table, idx3, w2).reshape(N_BOXES, 49, C)
    return jnp.transpose(out, (0, 2, 1)).reshape(N_BOXES, C, 7, 7)


# speedup vs baseline: 1.2626x; 1.1455x over previous
import functools

import jax
import jax.numpy as jnp
import numpy as np
from jax import lax
from jax.experimental import pallas as pl
from jax.experimental.pallas import tpu as pltpu
from jax.experimental.pallas import tpu_sc as plsc

N_BOXES = 1000
N_PAD = 1024
C = 256
K = 784
NCHUNK = 7
ROWS = 112
BINS_PER_CHUNK = 7
OUT_FLAT = C * 49
TAB_ROWS = 53125
NC, NS = 2, 16
NW = NC * NS
LANES = 16

_HS = (200, 100, 50, 25)
_BASES = (0, 40000, 50000, 52500)
_SCALES = (0.25, 0.125, 0.0625, 0.03125)


def _prep_body(boxes_ref, fy_ref, fx_ref, cy_ref, cx_ref, idx_ref, w_ref):
    x1b = boxes_ref[:, 0:1]
    y1b = boxes_ref[:, 1:2]
    x2b = boxes_ref[:, 2:3]
    y2b = boxes_ref[:, 3:4]
    area = (x2b - x1b) * (y2b - y1b)
    s = jnp.sqrt(area)
    lvl = jnp.floor(4.0 + jnp.log2(s / 224.0) + 1e-6)
    lvl = jnp.clip(lvl, 2.0, 5.0)
    l = (lvl - 2.0).astype(jnp.int32)
    scale = jnp.where(l == 0, _SCALES[0],
                      jnp.where(l == 1, _SCALES[1],
                                jnp.where(l == 2, _SCALES[2], _SCALES[3])))
    H = jnp.where(l == 0, _HS[0],
                  jnp.where(l == 1, _HS[1], jnp.where(l == 2, _HS[2], _HS[3])))
    base = jnp.where(l == 0, _BASES[0],
                     jnp.where(l == 1, _BASES[1],
                               jnp.where(l == 2, _BASES[2], _BASES[3])))
    x1 = x1b * scale
    y1 = y1b * scale
    x2 = x2b * scale
    y2 = y2b * scale
    bin_w = jnp.maximum(x2 - x1, 1.0) / 7.0
    bin_h = jnp.maximum(y2 - y1, 1.0) / 7.0

    fy = fy_ref[0:1, :]
    fx = fx_ref[0:1, :]
    cy = cy_ref[0:1, :]
    cx = cx_ref[0:1, :]

    yy = y1 + fy * bin_h
    xx = x1 + fx * bin_w
    Hf = H.astype(jnp.float32)
    m = ((yy > -1.0) & (yy < Hf) & (xx > -1.0) & (xx < Hf)).astype(jnp.float32)
    yc = jnp.clip(yy, 0.0, Hf - 1.0)
    xc = jnp.clip(xx, 0.0, Hf - 1.0)
    y0f = jnp.floor(yc)
    x0f = jnp.floor(xc)
    y0 = y0f.astype(jnp.int32)
    x0 = x0f.astype(jnp.int32)
    y1i = jnp.minimum(y0 + 1, H - 1)
    x1i = jnp.minimum(x0 + 1, H - 1)
    ly = yc - y0f
    lx = xc - x0f
    wy = jnp.where(cy == 0, 1.0 - ly, ly)
    wx = jnp.where(cx == 0, 1.0 - lx, lx)
    w_ref[...] = wy * wx * m * 0.25
    Y = jnp.where(cy == 0, y0, y1i)
    X = jnp.where(cx == 0, x0, x1i)
    idx_ref[...] = base + Y * H + X


def _prep_idx_w(boxes_p, fy, fx, cy, cx):
    grid = 8
    blk = N_PAD // grid
    return pl.pallas_call(
        _prep_body,
        grid=(grid,),
        in_specs=[
            pl.BlockSpec((blk, 4), lambda i: (i, 0)),
            pl.BlockSpec((8, K), lambda i: (0, 0)),
            pl.BlockSpec((8, K), lambda i: (0, 0)),
            pl.BlockSpec((8, K), lambda i: (0, 0)),
            pl.BlockSpec((8, K), lambda i: (0, 0)),
        ],
        out_specs=[
            pl.BlockSpec((blk, K), lambda i: (i, 0)),
            pl.BlockSpec((blk, K), lambda i: (i, 0)),
        ],
        out_shape=[
            jax.ShapeDtypeStruct((N_PAD, K), jnp.int32),
            jax.ShapeDtypeStruct((N_PAD, K), jnp.float32),
        ],
    )(boxes_p, fy, fx, cy, cx)


def _sc_body(table_hbm, idx_hbm, w_hbm, out_hbm, idx_v, w_v, rows_v, out_v,
             sem, isem, osem):
    wid = lax.axis_index("s") * NC + lax.axis_index("c")
    nb = jnp.where(wid < N_BOXES - 31 * NW, 32, 31)

    pltpu.async_copy(idx_hbm.at[wid], idx_v.at[0], isem.at[0])
    pltpu.async_copy(w_hbm.at[wid], w_v.at[0], isem.at[0])
    pltpu.make_async_copy(idx_hbm.at[wid], idx_v.at[0], isem.at[0]).wait()
    pltpu.make_async_copy(w_hbm.at[wid], w_v.at[0], isem.at[0]).wait()
    pltpu.async_copy(table_hbm.at[idx_v.at[0, 0]], rows_v.at[0], sem.at[0])

    @pl.loop(0, nb)
    def _box(i):
        cur = i % 2
        n = i * NW + wid

        @pl.when(i + 1 < nb)
        def _pref_idx():
            pltpu.async_copy(idx_hbm.at[n + NW], idx_v.at[1 - cur],
                             isem.at[1 - cur])
            pltpu.async_copy(w_hbm.at[n + NW], w_v.at[1 - cur],
                             isem.at[1 - cur])

        @pl.when(i >= 2)
        def _drain_out():
            pltpu.make_async_copy(out_v.at[cur], out_hbm.at[n - 2 * NW],
                                  osem.at[cur]).wait()

        @pl.loop(0, NCHUNK)
        def _chunk(c):
            s = (7 * i + c) % 2

            @pl.when(c + 1 < NCHUNK)
            def _prefetch():
                pltpu.async_copy(table_hbm.at[idx_v.at[cur, c + 1]],
                                 rows_v.at[1 - s], sem.at[1 - s])

            @pl.when((c + 1 == NCHUNK) & (i + 1 < nb))
            def _prefetch_next_box():
                pltpu.make_async_copy(idx_hbm.at[n + NW], idx_v.at[1 - cur],
                                      isem.at[1 - cur]).wait()
                pltpu.make_async_copy(w_hbm.at[n + NW], w_v.at[1 - cur],
                                      isem.at[1 - cur]).wait()
                pltpu.async_copy(table_hbm.at[idx_v.at[1 - cur, 0]],
                                 rows_v.at[1 - s], sem.at[1 - s])

            pltpu.make_async_copy(table_hbm.at[idx_v.at[cur, c]],
                                  rows_v.at[s], sem.at[s]).wait()

            @pl.loop(0, BINS_PER_CHUNK)
            def _bin(lb):
                g = c * BINS_PER_CHUNK + lb
                kbase = c * ROWS + lb * 16
                w16 = w_v[cur, pl.ds(kbase, LANES)]
                accs = [jnp.zeros((LANES,), jnp.float32) for _ in range(16)]
                for j in range(16):
                    wvec = jnp.take_along_axis(
                        w16, jnp.full((LANES,), j, jnp.int32), axis=0,
                        mode="promise_in_bounds")
                    for t2 in range(8):
                        xi = rows_v[s, lb * 16 + j, pl.ds(16 * t2, 16)]
                        a = lax.bitcast_convert_type(xi << 16, jnp.float32)
                        b = lax.bitcast_convert_type(xi, jnp.float32)
                        accs[t2] = accs[t2] + wvec * a
                        accs[8 + t2] = accs[8 + t2] + wvec * b
                for t in range(16):
                    out_v[cur, g, pl.ds(16 * t, LANES)] = accs[t]

        pltpu.async_copy(out_v.at[cur], out_hbm.at[n], osem.at[cur])

    pltpu.make_async_copy(out_v.at[nb % 2],
                          out_hbm.at[(nb - 2) * NW + wid],
                          osem.at[nb % 2]).wait()
    pltpu.make_async_copy(out_v.at[(nb - 1) % 2],
                          out_hbm.at[(nb - 1) * NW + wid],
                          osem.at[(nb - 1) % 2]).wait()


def _sc_gather(table, idx3, w2):
    mesh = plsc.VectorSubcoreMesh(core_axis_name="c", subcore_axis_name="s")
    fn = functools.partial(
        pl.kernel,
        out_type=jax.ShapeDtypeStruct((N_BOXES, 49, C), jnp.float32),
        mesh=mesh,
        scratch_types=[
            pltpu.VMEM((2, NCHUNK, ROWS), jnp.int32),
            pltpu.VMEM((2, K), jnp.float32),
            pltpu.VMEM((2, ROWS, C // 2), jnp.int32),
            pltpu.VMEM((2, 49, C), jnp.float32),
            pltpu.SemaphoreType.DMA((2,)),
            pltpu.SemaphoreType.DMA((2,)),
            pltpu.SemaphoreType.DMA((2,)),
        ],
    )(_sc_body)
    return fn(table, idx3, w2)


def _static_grids():
    k = np.arange(K)
    b = k // 16
    py = b // 7
    px = b % 7
    j = k % 16
    samp = j // 4
    iy = samp // 2
    ix = samp % 2
    corner = j % 4
    cy = corner // 2
    cx = corner % 2
    fy = (py + (iy + 0.5) / 2.0).astype(np.float32)
    fx = (px + (ix + 0.5) / 2.0).astype(np.float32)
    tile = lambda a: jnp.asarray(np.tile(a[None, :], (8, 1)))
    return (tile(fy), tile(fx), tile(cy.astype(np.int32)),
            tile(cx.astype(np.int32)))


@jax.jit
def kernel(feat0, feat1, feat2, feat3, boxes):
    table = jnp.concatenate(
        [jnp.transpose(f[0], (1, 2, 0)).reshape(-1, C)
         for f in (feat0, feat1, feat2, feat3)], axis=0)
    lo = jax.lax.bitcast_convert_type(
        table[:, :128].astype(jnp.bfloat16), jnp.uint16).astype(jnp.uint32)
    hi = jax.lax.bitcast_convert_type(
        table[:, 128:].astype(jnp.bfloat16), jnp.uint16).astype(jnp.uint32)
    table = jax.lax.bitcast_convert_type(lo | (hi << 16), jnp.int32)
    boxes_p = jnp.zeros((N_PAD, 4), jnp.float32).at[:N_BOXES].set(boxes)
    fy, fx, cy, cx = _static_grids()
    idx, w = _prep_idx_w(boxes_p, fy, fx, cy, cx)
    idx3 = idx[:N_BOXES].reshape(N_BOXES, NCHUNK, ROWS)
    w2 = w[:N_BOXES]
    out = _sc_gather(table, idx3, w2).reshape(N_BOXES, 49, C)
    return jnp.transpose(out, (0, 2, 1)).reshape(N_BOXES, C, 7, 7)
